# baseline (speedup 1.0000x reference)
import functools

import jax
import jax.numpy as jnp
import numpy as np
from jax import lax
from jax.experimental import pallas as pl
from jax.experimental.pallas import tpu as pltpu
from jax.experimental.pallas import tpu_sc as plsc

_NUM_EMB = 256
_IN_CH = 96
_MIN_SCALE = 2.5
_MAX_SCALE = 3.5

_B = 16
_HW = 56 * 56
_SP = _B * _HW
_N = _SP * _IN_CH

_LANES = 16
_NCHG = _IN_CH // _LANES
_NW = 32
_RPW = _SP // _NW
_RCH = 16
_NCHUNK = _RPW // _RCH
_NPAIR = _NCHUNK // 2


def _stats_body(x_ref, scale_ref, s_ref):
    xv = x_ref[...]
    s1 = jnp.sum(xv)
    s2 = jnp.sum(xv * xv)
    mx = jnp.max(jnp.abs(xv))
    n = jnp.float32(_N)
    var = (s2 - s1 * (s1 / n)) / (n - 1.0)
    std = jnp.sqrt(var)
    min_s = jnp.float32(_MIN_SCALE * 0.9) + (2.0 * std) * jnp.float32(1.0 - 0.9)
    max_s = jnp.float32(_MAX_SCALE * 0.9) + mx * jnp.float32(1.0 - 0.9)
    s_ref[...] = jnp.minimum(jnp.maximum(scale_ref[...], min_s), max_s)


_stats = pl.pallas_call(
    _stats_body,
    out_shape=jax.ShapeDtypeStruct((1, _IN_CH), jnp.float32),
    compiler_params=pltpu.CompilerParams(vmem_limit_bytes=60000 * 1024),
)


def _remap_body(x_hbm, s_hbm, tab_hbm, out_hbm, tab_v, s_v, in_v0, in_v1,
                out_v0, out_v1, sem_in0, sem_in1, sem_out0, sem_out1):
    in_bufs = (in_v0, in_v1)
    out_bufs = (out_v0, out_v1)
    sems_in = (sem_in0, sem_in1)
    sems_out = (sem_out0, sem_out1)
    wid = lax.axis_index("s") * 2 + lax.axis_index("c")
    row0 = wid * _RPW
    pltpu.sync_copy(tab_hbm, tab_v)
    pltpu.sync_copy(s_hbm, s_v)

    svs = [s_v[pl.ds(_LANES * j, _LANES)] for j in range(_NCHG)]
    lane = lax.iota(jnp.int32, _LANES).astype(jnp.float32)
    offvs = [
        (lane + jnp.float32(_LANES * j)) * jnp.float32(_NUM_EMB)
        for j in range(_NCHG)
    ]

    def start_in(ch, buf):
        start = pl.multiple_of(row0 + ch * _RCH, 8)
        pltpu.async_copy(
            x_hbm.at[pl.ds(start, _RCH)], in_bufs[buf], sems_in[buf])

    def start_out(ch, buf):
        start = pl.multiple_of(row0 + ch * _RCH, 8)
        pltpu.async_copy(
            out_bufs[buf], out_hbm.at[pl.ds(start, _RCH)], sems_out[buf])

    def wait_in(buf):
        pltpu.make_async_copy(
            x_hbm.at[pl.ds(0, _RCH)], in_bufs[buf], sems_in[buf]).wait()

    def wait_out(buf):
        pltpu.make_async_copy(
            out_bufs[buf], out_hbm.at[pl.ds(0, _RCH)], sems_out[buf]).wait()

    def compute(buf):
        in_v = in_bufs[buf]
        out_v = out_bufs[buf]

        @plsc.parallel_loop(0, _RCH, 1, unroll=1)
        def _(r):
            for j in range(_NCHG):
                sv = svs[j]
                xv = in_v[r, pl.ds(_LANES * j, _LANES)]
                xc = jnp.minimum(jnp.maximum(xv, -sv), sv)
                t0 = xc / sv
                t = ((t0 + 1.0) * 0.5) * 255.0 + offvs[j]
                li = t.astype(jnp.int32)
                lf = li.astype(jnp.float32)
                frac = t - lf
                ui = li + (t > lf).astype(jnp.int32)
                lv = plsc.load_gather(tab_v, [li])
                uv = plsc.load_gather(tab_v, [ui])
                out_v[r, pl.ds(_LANES * j, _LANES)] = uv + frac * (lv - uv)

    start_in(0, 0)

    def pair(k, carry):
        ch0 = 2 * k
        start_in(ch0 + 1, 1)
        wait_in(0)

        @pl.when(k > 0)
        def _():
            wait_out(0)

        compute(0)
        start_out(ch0, 0)

        @pl.when(k < _NPAIR - 1)
        def _():
            start_in(ch0 + 2, 0)

        wait_in(1)

        @pl.when(k > 0)
        def _():
            wait_out(1)

        compute(1)
        start_out(ch0 + 1, 1)
        return carry

    lax.fori_loop(0, _NPAIR, pair, None)
    wait_out(0)
    wait_out(1)


@functools.cache
def _build_remap():
    mesh = plsc.VectorSubcoreMesh(core_axis_name="c", subcore_axis_name="s")
    return pl.kernel(
        _remap_body,
        out_type=jax.ShapeDtypeStruct((_SP, _IN_CH), jnp.float32),
        mesh=mesh,
        compiler_params=pltpu.CompilerParams(needs_layout_passes=False),
        scratch_types=[
            pltpu.VMEM((_NUM_EMB * _IN_CH,), jnp.float32),
            pltpu.VMEM((_IN_CH,), jnp.float32),
            pltpu.VMEM((_RCH, _IN_CH), jnp.float32),
            pltpu.VMEM((_RCH, _IN_CH), jnp.float32),
            pltpu.VMEM((_RCH, _IN_CH), jnp.float32),
            pltpu.VMEM((_RCH, _IN_CH), jnp.float32),
            pltpu.SemaphoreType.DMA,
            pltpu.SemaphoreType.DMA,
            pltpu.SemaphoreType.DMA,
            pltpu.SemaphoreType.DMA,
        ],
    )


def kernel(x, scale, table):
    xt = x.transpose(0, 2, 3, 1).reshape(_SP, _IN_CH)
    s = _stats(xt, scale.reshape(1, _IN_CH))
    _remap = _build_remap()
    out = _remap(xt, s.reshape(_IN_CH), table.reshape(-1))
    return out.reshape(_B, 56, 56, _IN_CH).transpose(0, 3, 1, 2)


# speedup vs baseline: 1.2644x; 1.2644x over previous
import functools

import jax
import jax.numpy as jnp
import numpy as np
from jax import lax
from jax.experimental import pallas as pl
from jax.experimental.pallas import tpu as pltpu
from jax.experimental.pallas import tpu_sc as plsc

_NUM_EMB = 256
_IN_CH = 96
_MIN_SCALE = 2.5
_MAX_SCALE = 3.5

_B = 16
_HW = 56 * 56
_SP = _B * _HW
_N = _SP * _IN_CH

_LANES = 16
_NCHG = _IN_CH // _LANES
_NW = 32
_RPW = _SP // _NW
_RCH = 56
_NCHUNK = _RPW // _RCH
_NPAIR = _NCHUNK // 2


def _stats_body(x_ref, scale_ref, s_ref):
    xv = x_ref[...]
    s1 = jnp.sum(xv)
    s2 = jnp.sum(xv * xv)
    mx = jnp.max(jnp.abs(xv))
    n = jnp.float32(_N)
    var = (s2 - s1 * (s1 / n)) / (n - 1.0)
    std = jnp.sqrt(var)
    min_s = jnp.float32(_MIN_SCALE * 0.9) + (2.0 * std) * jnp.float32(1.0 - 0.9)
    max_s = jnp.float32(_MAX_SCALE * 0.9) + mx * jnp.float32(1.0 - 0.9)
    s_ref[...] = jnp.minimum(jnp.maximum(scale_ref[...], min_s), max_s)


_stats = pl.pallas_call(
    _stats_body,
    out_shape=jax.ShapeDtypeStruct((1, _IN_CH), jnp.float32),
    compiler_params=pltpu.CompilerParams(vmem_limit_bytes=60000 * 1024),
)


def _remap_body(x_hbm, s_hbm, tab_hbm, out_hbm, tab_v, s_v, in_v0, in_v1,
                out_v0, out_v1, sem_in0, sem_in1, sem_out0, sem_out1):
    in_bufs = (in_v0, in_v1)
    out_bufs = (out_v0, out_v1)
    sems_in = (sem_in0, sem_in1)
    sems_out = (sem_out0, sem_out1)
    wid = lax.axis_index("s") * 2 + lax.axis_index("c")
    row0 = wid * _RPW
    pltpu.sync_copy(tab_hbm, tab_v)
    pltpu.sync_copy(s_hbm, s_v)

    svs = [s_v[pl.ds(_LANES * j, _LANES)] for j in range(_NCHG)]
    lane = lax.iota(jnp.int32, _LANES).astype(jnp.float32)
    offvs = [
        (lane + jnp.float32(_LANES * j)) * jnp.float32(_NUM_EMB)
        for j in range(_NCHG)
    ]

    def start_in(ch, buf):
        start = pl.multiple_of(row0 + ch * _RCH, 8)
        pltpu.async_copy(
            x_hbm.at[pl.ds(start, _RCH)], in_bufs[buf], sems_in[buf])

    def start_out(ch, buf):
        start = pl.multiple_of(row0 + ch * _RCH, 8)
        pltpu.async_copy(
            out_bufs[buf], out_hbm.at[pl.ds(start, _RCH)], sems_out[buf])

    def wait_in(buf):
        pltpu.make_async_copy(
            x_hbm.at[pl.ds(0, _RCH)], in_bufs[buf], sems_in[buf]).wait()

    def wait_out(buf):
        pltpu.make_async_copy(
            out_bufs[buf], out_hbm.at[pl.ds(0, _RCH)], sems_out[buf]).wait()

    def compute(buf):
        in_v = in_bufs[buf]
        out_v = out_bufs[buf]

        @plsc.parallel_loop(0, _RCH, 1, unroll=1)
        def _(r):
            for j in range(_NCHG):
                sv = svs[j]
                xv = in_v[r, pl.ds(_LANES * j, _LANES)]
                xc = jnp.minimum(jnp.maximum(xv, -sv), sv)
                t0 = xc / sv
                t = ((t0 + 1.0) * 0.5) * 255.0 + offvs[j]
                li = t.astype(jnp.int32)
                lf = li.astype(jnp.float32)
                frac = t - lf
                ui = li + (t > lf).astype(jnp.int32)
                lv = plsc.load_gather(tab_v, [li])
                uv = plsc.load_gather(tab_v, [ui])
                out_v[r, pl.ds(_LANES * j, _LANES)] = uv + frac * (lv - uv)

    start_in(0, 0)

    def pair(k, carry):
        ch0 = 2 * k
        start_in(ch0 + 1, 1)
        wait_in(0)

        @pl.when(k > 0)
        def _():
            wait_out(0)

        compute(0)
        start_out(ch0, 0)

        @pl.when(k < _NPAIR - 1)
        def _():
            start_in(ch0 + 2, 0)

        wait_in(1)

        @pl.when(k > 0)
        def _():
            wait_out(1)

        compute(1)
        start_out(ch0 + 1, 1)
        return carry

    lax.fori_loop(0, _NPAIR, pair, None)
    wait_out(0)
    wait_out(1)


@functools.cache
def _build_remap():
    mesh = plsc.VectorSubcoreMesh(core_axis_name="c", subcore_axis_name="s")
    return pl.kernel(
        _remap_body,
        out_type=jax.ShapeDtypeStruct((_SP, _IN_CH), jnp.float32),
        mesh=mesh,
        compiler_params=pltpu.CompilerParams(needs_layout_passes=False),
        scratch_types=[
            pltpu.VMEM((_NUM_EMB * _IN_CH,), jnp.float32),
            pltpu.VMEM((_IN_CH,), jnp.float32),
            pltpu.VMEM((_RCH, _IN_CH), jnp.float32),
            pltpu.VMEM((_RCH, _IN_CH), jnp.float32),
            pltpu.VMEM((_RCH, _IN_CH), jnp.float32),
            pltpu.VMEM((_RCH, _IN_CH), jnp.float32),
            pltpu.SemaphoreType.DMA,
            pltpu.SemaphoreType.DMA,
            pltpu.SemaphoreType.DMA,
            pltpu.SemaphoreType.DMA,
        ],
    )


def kernel(x, scale, table):
    xt = x.transpose(0, 2, 3, 1).reshape(_SP, _IN_CH)
    s = _stats(xt, scale.reshape(1, _IN_CH))
    _remap = _build_remap()
    out = _remap(xt, s.reshape(_IN_CH), table.reshape(-1))
    return out.reshape(_B, 56, 56, _IN_CH).transpose(0, 3, 1, 2)


# speedup vs baseline: 1.2653x; 1.0007x over previous
import functools

import jax
import jax.numpy as jnp
import numpy as np
from jax import lax
from jax.experimental import pallas as pl
from jax.experimental.pallas import tpu as pltpu
from jax.experimental.pallas import tpu_sc as plsc

_NUM_EMB = 256
_IN_CH = 96
_MIN_SCALE = 2.5
_MAX_SCALE = 3.5

_B = 16
_HW = 56 * 56
_SP = _B * _HW
_N = _SP * _IN_CH

_LANES = 16
_NCHG = _IN_CH // _LANES
_NW = 32
_RPW = _SP // _NW
_RCH = 56
_NCHUNK = _RPW // _RCH
_NPAIR = _NCHUNK // 2


def _stats_body(x_ref, scale_ref, s_ref):
    xv = x_ref[...]
    s1 = jnp.sum(xv)
    s2 = jnp.sum(xv * xv)
    mx = jnp.max(jnp.abs(xv))
    n = jnp.float32(_N)
    var = (s2 - s1 * (s1 / n)) / (n - 1.0)
    std = jnp.sqrt(var)
    min_s = jnp.float32(_MIN_SCALE * 0.9) + (2.0 * std) * jnp.float32(1.0 - 0.9)
    max_s = jnp.float32(_MAX_SCALE * 0.9) + mx * jnp.float32(1.0 - 0.9)
    s_ref[...] = jnp.minimum(jnp.maximum(scale_ref[...], min_s), max_s)


_stats = pl.pallas_call(
    _stats_body,
    out_shape=jax.ShapeDtypeStruct((1, _IN_CH), jnp.float32),
    compiler_params=pltpu.CompilerParams(vmem_limit_bytes=60000 * 1024),
)


def _remap_body(x_hbm, s_hbm, tab_hbm, out_hbm, tab_v, s_v, in_v0, in_v1,
                out_v0, out_v1, sem_in0, sem_in1, sem_out0, sem_out1):
    in_bufs = (in_v0, in_v1)
    out_bufs = (out_v0, out_v1)
    sems_in = (sem_in0, sem_in1)
    sems_out = (sem_out0, sem_out1)
    wid = lax.axis_index("s") * 2 + lax.axis_index("c")
    row0 = wid * _RPW
    pltpu.sync_copy(tab_hbm, tab_v)
    pltpu.sync_copy(s_hbm, s_v)

    svs = [s_v[pl.ds(_LANES * j, _LANES)] for j in range(_NCHG)]
    lane = lax.iota(jnp.int32, _LANES).astype(jnp.float32)
    offvs = [
        (lane + jnp.float32(_LANES * j)) * jnp.float32(_NUM_EMB)
        for j in range(_NCHG)
    ]

    def start_in(ch, buf):
        start = pl.multiple_of(row0 + ch * _RCH, 8)
        pltpu.async_copy(
            x_hbm.at[pl.ds(start, _RCH)], in_bufs[buf], sems_in[buf])

    def start_out(ch, buf):
        start = pl.multiple_of(row0 + ch * _RCH, 8)
        pltpu.async_copy(
            out_bufs[buf], out_hbm.at[pl.ds(start, _RCH)], sems_out[buf])

    def wait_in(buf):
        pltpu.make_async_copy(
            x_hbm.at[pl.ds(0, _RCH)], in_bufs[buf], sems_in[buf]).wait()

    def wait_out(buf):
        pltpu.make_async_copy(
            out_bufs[buf], out_hbm.at[pl.ds(0, _RCH)], sems_out[buf]).wait()

    def compute(buf):
        in_v = in_bufs[buf]
        out_v = out_bufs[buf]

        @plsc.parallel_loop(0, _RCH, 1, unroll=1)
        def _(r):
            for j in range(_NCHG):
                sv = svs[j]
                xv = in_v[r, pl.ds(_LANES * j, _LANES)]
                xc = jnp.clip(xv, -sv, sv)
                t0 = xc / sv
                t = ((t0 + 1.0) * 0.5) * 255.0 + offvs[j]
                li = t.astype(jnp.int32)
                lf = li.astype(jnp.float32)
                frac = t - lf
                ui = li + (t > lf).astype(jnp.int32)
                lv = plsc.load_gather(tab_v, [li])
                uv = plsc.load_gather(tab_v, [ui])
                out_v[r, pl.ds(_LANES * j, _LANES)] = uv + frac * (lv - uv)

    start_in(0, 0)

    def pair(k, carry):
        ch0 = 2 * k
        start_in(ch0 + 1, 1)
        wait_in(0)

        @pl.when(k > 0)
        def _():
            wait_out(0)

        compute(0)
        start_out(ch0, 0)

        @pl.when(k < _NPAIR - 1)
        def _():
            start_in(ch0 + 2, 0)

        wait_in(1)

        @pl.when(k > 0)
        def _():
            wait_out(1)

        compute(1)
        start_out(ch0 + 1, 1)
        return carry

    lax.fori_loop(0, _NPAIR, pair, None)
    wait_out(0)
    wait_out(1)


@functools.cache
def _build_remap():
    mesh = plsc.VectorSubcoreMesh(core_axis_name="c", subcore_axis_name="s")
    return pl.kernel(
        _remap_body,
        out_type=jax.ShapeDtypeStruct((_SP, _IN_CH), jnp.float32),
        mesh=mesh,
        compiler_params=pltpu.CompilerParams(needs_layout_passes=False),
        scratch_types=[
            pltpu.VMEM((_NUM_EMB * _IN_CH,), jnp.float32),
            pltpu.VMEM((_IN_CH,), jnp.float32),
            pltpu.VMEM((_RCH, _IN_CH), jnp.float32),
            pltpu.VMEM((_RCH, _IN_CH), jnp.float32),
            pltpu.VMEM((_RCH, _IN_CH), jnp.float32),
            pltpu.VMEM((_RCH, _IN_CH), jnp.float32),
            pltpu.SemaphoreType.DMA,
            pltpu.SemaphoreType.DMA,
            pltpu.SemaphoreType.DMA,
            pltpu.SemaphoreType.DMA,
        ],
    )


def kernel(x, scale, table):
    xt = x.transpose(0, 2, 3, 1).reshape(_SP, _IN_CH)
    s = _stats(xt, scale.reshape(1, _IN_CH))
    _remap = _build_remap()
    out = _remap(xt, s.reshape(_IN_CH), table.reshape(-1))
    return out.reshape(_B, 56, 56, _IN_CH).transpose(0, 3, 1, 2)
